# Initial kernel scaffold; baseline (speedup 1.0000x reference)
import jax
import jax.numpy as jnp
from jax.experimental import pallas as pl


def kernel(x, edge_index, batch, W1, b1, W2, b2, Wc1, bc1, Wc2, bc2):
    raise NotImplementedError("write your pallas kernel here")


# speedup vs baseline: 21.3757x; 21.3757x over previous
import functools

import jax
import jax.numpy as jnp
from jax import lax
from jax.experimental import pallas as pl
from jax.experimental.pallas import tpu as pltpu
from jax.experimental.pallas import tpu_sc as plsc

NN = 10000
EE = 320000
NCORE = 2
NSUB = 16
NW = NCORE * NSUB
EPT = EE // NW
CH = 40
NCHUNK = 5
BURST = CH * NCHUNK
NBURST = EPT // BURST
NPAD = 10240
RPT = NPAD // NSUB
RB = 128
RPT_DEG = RPT
CH_D = 80
NCH_D = 5
NB_D = EPT // (CH_D * NCH_D)


def _mesh():
    return plsc.VectorSubcoreMesh(
        core_axis_name="c", subcore_axis_name="s",
        num_cores=NCORE, num_subcores=NSUB)


def _zero_fill_1d(ref, nwords):
    z = jnp.zeros((16,), jnp.float32)

    def body(i, carry):
        ref[pl.ds(i * 16, 16)] = z
        return carry

    lax.fori_loop(0, nwords // 16, body, 0)


def _zero_fill_2d(ref, rows, cols):
    z = jnp.zeros((16,), jnp.float32)
    c16 = cols // 16

    def body(i, carry):
        r = i // c16
        c = i % c16
        ref[r, pl.ds(c * 16, 16)] = z
        return carry

    lax.fori_loop(0, rows * c16, body, 0)


def _sc_deg(dst):
    @functools.partial(
        pl.kernel,
        out_type=jax.ShapeDtypeStruct((NCORE, NPAD), jnp.float32),
        mesh=_mesh(),
        scratch_types=[
            pltpu.VMEM((NCH_D, CH_D), jnp.int32),
            pltpu.VMEM((CH_D,), jnp.float32),
            pltpu.VMEM((RPT_DEG,), jnp.float32),
            pltpu.VMEM_SHARED((NPAD,), jnp.float32),
            pltpu.SemaphoreType.DMA,
            pltpu.SemaphoreType.DMA,
        ],
    )
    def deg_kernel(dst_hbm, out_hbm, dv, ones_v, zb, acc, isem, ssem):
        cid = lax.axis_index("c")
        sid = lax.axis_index("s")
        wid = sid * NCORE + cid
        one = jnp.ones((16,), jnp.float32)
        for k in range(CH_D // 16):
            ones_v[pl.ds(k * 16, 16)] = one
        _zero_fill_1d(zb, RPT_DEG)
        pltpu.sync_copy(zb, acc.at[pl.ds(sid * RPT_DEG, RPT_DEG)])
        plsc.subcore_barrier()

        base = wid * EPT

        def burst(b, carry):
            off = pl.multiple_of(base + b * CH_D * NCH_D, 8)
            hl = [pltpu.async_copy(dst_hbm.at[pl.ds(off + k * CH_D, CH_D)],
                                   dv.at[k], isem)
                  for k in range(NCH_D)]
            for h in hl:
                h.wait()
            hs = [pltpu.async_copy(ones_v, acc.at[dv.at[k]], ssem, add=True)
                  for k in range(NCH_D)]
            for h in hs:
                h.wait()
            return carry

        lax.fori_loop(0, NB_D, burst, 0)
        plsc.subcore_barrier()
        pltpu.sync_copy(acc.at[pl.ds(sid * RPT_DEG, RPT_DEG)], zb)
        pltpu.sync_copy(zb, out_hbm.at[cid, pl.ds(sid * RPT_DEG, RPT_DEG)])

    return deg_kernel(dst)


DP = 128


def _sc_agg(y, src, dst):
    @functools.partial(
        pl.kernel,
        out_type=jax.ShapeDtypeStruct((NCORE, NPAD, DP), jnp.float32),
        mesh=_mesh(),
        scratch_types=[
            pltpu.VMEM((NCHUNK, CH), jnp.int32),
            pltpu.VMEM((NCHUNK, CH), jnp.int32),
            pltpu.VMEM((BURST, DP), jnp.float32),
            pltpu.VMEM_SHARED((NPAD, DP), jnp.float32),
            pltpu.SemaphoreType.DMA,
            pltpu.SemaphoreType.DMA,
            pltpu.SemaphoreType.DMA,
        ],
    )
    def agg_kernel(y_hbm, src_hbm, dst_hbm, out_hbm,
                   sv, dv, rows, acc, isem, gsem, ssem):
        cid = lax.axis_index("c")
        sid = lax.axis_index("s")
        wid = sid * NCORE + cid
        stage = rows.at[pl.ds(0, RB)]
        _zero_fill_2d(rows, RB, DP)
        r0 = sid * RPT
        for j in range(RPT // RB):
            pltpu.sync_copy(stage, acc.at[pl.ds(r0 + j * RB, RB)])
        plsc.subcore_barrier()

        base = wid * EPT

        def burst(b, carry):
            off = pl.multiple_of(base + b * BURST, 8)
            hl = [pltpu.async_copy(src_hbm.at[pl.ds(off + k * CH, CH)],
                                   sv.at[k], isem)
                  for k in range(NCHUNK)]
            hl += [pltpu.async_copy(dst_hbm.at[pl.ds(off + k * CH, CH)],
                                    dv.at[k], isem)
                   for k in range(NCHUNK)]
            for h in hl:
                h.wait()
            hg = [pltpu.async_copy(y_hbm.at[sv.at[k]],
                                   rows.at[pl.ds(k * CH, CH)], gsem)
                  for k in range(NCHUNK)]
            for h in hg:
                h.wait()
            hs = [pltpu.async_copy(rows.at[pl.ds(k * CH, CH)],
                                   acc.at[dv.at[k]], ssem, add=True)
                  for k in range(NCHUNK)]
            for h in hs:
                h.wait()
            return carry

        lax.fori_loop(0, NBURST, burst, 0)
        plsc.subcore_barrier()
        for j in range(RPT // RB):
            pltpu.sync_copy(acc.at[pl.ds(r0 + j * RB, RB)], stage)
            pltpu.sync_copy(stage, out_hbm.at[cid, pl.ds(r0 + j * RB, RB)])

    return agg_kernel(y, src, dst)


RBLK = 1000


def _tc1_body(deg_ref, x_ref, w_ref, s_ref, y_ref):
    deg = deg_ref[:, 0] + deg_ref[:, 1] + 1.0
    s = lax.rsqrt(deg)[:, None]
    s_ref[...] = s
    y_ref[...] = jnp.dot(x_ref[...] * s, w_ref[...],
                         preferred_element_type=jnp.float32)


def _tc1(degp, x, W1):
    return pl.pallas_call(
        _tc1_body,
        grid=(NN // RBLK,),
        in_specs=[
            pl.BlockSpec((RBLK, NCORE), lambda i: (i, 0)),
            pl.BlockSpec((RBLK, 128), lambda i: (i, 0)),
            pl.BlockSpec((128, 128), lambda i: (0, 0)),
        ],
        out_specs=[
            pl.BlockSpec((RBLK, 1), lambda i: (i, 0)),
            pl.BlockSpec((RBLK, 128), lambda i: (i, 0)),
        ],
        out_shape=[
            jax.ShapeDtypeStruct((NN, 1), jnp.float32),
            jax.ShapeDtypeStruct((NN, 128), jnp.float32),
        ],
    )(degp, x, W1)


def _tc2_body(p_ref, y1_ref, s_ref, b1_ref, w2_ref, y2_ref):
    s = s_ref[...]
    agg = p_ref[0][:, :64] + p_ref[1][:, :64] + y1_ref[...][:, :64]
    h1 = jnp.maximum(agg * s + b1_ref[...], 0.0)
    y2_ref[...] = jnp.dot(h1 * s, w2_ref[...],
                          preferred_element_type=jnp.float32)


def _tc2(p1, y1, s, b1, W2):
    return pl.pallas_call(
        _tc2_body,
        grid=(NN // RBLK,),
        in_specs=[
            pl.BlockSpec((NCORE, RBLK, DP), lambda i: (0, i, 0)),
            pl.BlockSpec((RBLK, DP), lambda i: (i, 0)),
            pl.BlockSpec((RBLK, 1), lambda i: (i, 0)),
            pl.BlockSpec((64,), lambda i: (0,)),
            pl.BlockSpec((64, 128), lambda i: (0, 0)),
        ],
        out_specs=pl.BlockSpec((RBLK, 128), lambda i: (i, 0)),
        out_shape=jax.ShapeDtypeStruct((NN, 128), jnp.float32),
    )(p1, y1, s, b1, W2)


def _tc3_body(p_ref, y2_ref, s_ref, b2_ref, wc1_ref, bc1_ref,
              wc2_ref, bc2_ref, o_ref):
    s = s_ref[...]
    agg = p_ref[0][:, :32] + p_ref[1][:, :32] + y2_ref[...][:, :32]
    h2 = jnp.maximum(agg * s + b2_ref[...], 0.0)
    h3 = jnp.maximum(
        jnp.dot(h2, wc1_ref[...], preferred_element_type=jnp.float32)
        + bc1_ref[...], 0.0)
    o_ref[...] = (jnp.dot(h3, wc2_ref[...], preferred_element_type=jnp.float32)
                  + bc2_ref[...])


def _tc3(p2, y2, s, b2, Wc1, bc1, Wc2, bc2):
    return pl.pallas_call(
        _tc3_body,
        grid=(NN // RBLK,),
        in_specs=[
            pl.BlockSpec((NCORE, RBLK, DP), lambda i: (0, i, 0)),
            pl.BlockSpec((RBLK, DP), lambda i: (i, 0)),
            pl.BlockSpec((RBLK, 1), lambda i: (i, 0)),
            pl.BlockSpec((32,), lambda i: (0,)),
            pl.BlockSpec((32, 16), lambda i: (0, 0)),
            pl.BlockSpec((16,), lambda i: (0,)),
            pl.BlockSpec((16, 10), lambda i: (0, 0)),
            pl.BlockSpec((10,), lambda i: (0,)),
        ],
        out_specs=pl.BlockSpec((RBLK, 10), lambda i: (i, 0)),
        out_shape=jax.ShapeDtypeStruct((NN, 10), jnp.float32),
    )(p2, y2, s, b2, Wc1, bc1, Wc2, bc2)


def kernel(x, edge_index, batch, W1, b1, W2, b2, Wc1, bc1, Wc2, bc2):
    src = edge_index[0]
    dst = edge_index[1]
    W1p = jnp.pad(W1, ((0, 0), (0, DP - W1.shape[1])))
    W2p = jnp.pad(W2, ((0, 0), (0, DP - W2.shape[1])))
    degp = _sc_deg(dst)[:, :NN].T
    s, y1 = _tc1(degp, x, W1p)
    p1 = _sc_agg(y1, src, dst)
    y2 = _tc2(p1, y1, s, b1, W2p)
    p2 = _sc_agg(y2, src, dst)
    return _tc3(p2, y2, s, b2, Wc1, bc1, Wc2, bc2)


# speedup vs baseline: 32.6840x; 1.5290x over previous
import functools

import jax
import jax.numpy as jnp
from jax import lax
from jax.experimental import pallas as pl
from jax.experimental.pallas import tpu as pltpu
from jax.experimental.pallas import tpu_sc as plsc

NN = 10000
EE = 320000
NCORE = 2
NSUB = 16
NW = NCORE * NSUB
EPT = EE // NW
GCH = 80
NPAD = 10240
RPT = NPAD // NSUB
RB = 128
RPT_DEG = RPT
CH_D = 80
NCH_D = 5
NB_D = EPT // (CH_D * NCH_D)


def _mesh():
    return plsc.VectorSubcoreMesh(
        core_axis_name="c", subcore_axis_name="s",
        num_cores=NCORE, num_subcores=NSUB)


def _zero_fill_1d(ref, nwords):
    z = jnp.zeros((16,), jnp.float32)

    def body(i, carry):
        ref[pl.ds(i * 16, 16)] = z
        return carry

    lax.fori_loop(0, nwords // 16, body, 0)


def _zero_fill_2d(ref, rows, cols):
    z = jnp.zeros((16,), jnp.float32)
    c16 = cols // 16

    def body(i, carry):
        r = i // c16
        c = i % c16
        ref[r, pl.ds(c * 16, 16)] = z
        return carry

    lax.fori_loop(0, rows * c16, body, 0)


def _sc_deg(dst):
    @functools.partial(
        pl.kernel,
        out_type=jax.ShapeDtypeStruct((NCORE, NPAD), jnp.float32),
        mesh=_mesh(),
        scratch_types=[
            pltpu.VMEM((NCH_D, CH_D), jnp.int32),
            pltpu.VMEM((CH_D,), jnp.float32),
            pltpu.VMEM((RPT_DEG,), jnp.float32),
            pltpu.VMEM_SHARED((NPAD,), jnp.float32),
            pltpu.SemaphoreType.DMA,
            pltpu.SemaphoreType.DMA,
        ],
    )
    def deg_kernel(dst_hbm, out_hbm, dv, ones_v, zb, acc, isem, ssem):
        cid = lax.axis_index("c")
        sid = lax.axis_index("s")
        wid = sid * NCORE + cid
        one = jnp.ones((16,), jnp.float32)
        for k in range(CH_D // 16):
            ones_v[pl.ds(k * 16, 16)] = one
        _zero_fill_1d(zb, RPT_DEG)
        pltpu.sync_copy(zb, acc.at[pl.ds(sid * RPT_DEG, RPT_DEG)])
        plsc.subcore_barrier()

        base = wid * EPT

        def burst(b, carry):
            off = pl.multiple_of(base + b * CH_D * NCH_D, 8)
            hl = [pltpu.async_copy(dst_hbm.at[pl.ds(off + k * CH_D, CH_D)],
                                   dv.at[k], isem)
                  for k in range(NCH_D)]
            for h in hl:
                h.wait()
            hs = [pltpu.async_copy(ones_v, acc.at[dv.at[k]], ssem, add=True)
                  for k in range(NCH_D)]
            for h in hs:
                h.wait()
            return carry

        lax.fori_loop(0, NB_D, burst, 0)
        plsc.subcore_barrier()
        pltpu.sync_copy(acc.at[pl.ds(sid * RPT_DEG, RPT_DEG)], zb)
        pltpu.sync_copy(zb, out_hbm.at[cid, pl.ds(sid * RPT_DEG, RPT_DEG)])

    return deg_kernel(dst)


DP = 128


def _sc_agg(y, src, dst):
    NC_T = EPT // GCH
    NI = 8
    NR = 4

    @functools.partial(
        pl.kernel,
        out_type=jax.ShapeDtypeStruct((NCORE, NPAD, DP), jnp.float32),
        mesh=_mesh(),
        scratch_types=[
            pltpu.VMEM((NI, GCH), jnp.int32),
            pltpu.VMEM((NI, GCH), jnp.int32),
            pltpu.VMEM((NR, GCH, DP), jnp.float32),
            pltpu.VMEM_SHARED((NPAD, DP), jnp.float32),
            pltpu.SemaphoreType.DMA,
            pltpu.SemaphoreType.DMA,
            pltpu.SemaphoreType.DMA,
        ],
    )
    def agg_kernel(y_hbm, src_hbm, dst_hbm, out_hbm,
                   sv, dv, rows, acc, isem, gsem, ssem):
        cid = lax.axis_index("c")
        sid = lax.axis_index("s")
        wid = sid * NCORE + cid
        base = wid * EPT

        def issue_idx(g, ig):
            off = pl.multiple_of(base + g * GCH, 8)
            pltpu.async_copy(src_hbm.at[pl.ds(off, GCH)], sv.at[ig], isem)
            pltpu.async_copy(dst_hbm.at[pl.ds(off, GCH)], dv.at[ig], isem)

        def wait_idx(ig):
            pltpu.make_async_copy(src_hbm.at[pl.ds(0, GCH)], sv.at[ig], isem).wait()
            pltpu.make_async_copy(dst_hbm.at[pl.ds(0, GCH)], dv.at[ig], isem).wait()

        def wait_rows(rg, sem):
            pltpu.make_async_copy(y_hbm.at[pl.ds(0, GCH)], rows.at[rg], sem).wait()

        LAST = NC_T - 1

        def step(g, ig, rg):
            wait_rows(rg, gsem)
            pltpu.async_copy(rows.at[rg], acc.at[dv.at[ig]], ssem, add=True)
            if not isinstance(g, int) or g >= 2:
                wait_rows((rg + 2) % NR, ssem)
            if not isinstance(g, int) or g + 2 <= LAST:
                ig2 = (ig + 2) % NI
                wait_idx(ig2)
                pltpu.async_copy(y_hbm.at[sv.at[ig2]],
                                 rows.at[(rg + 2) % NR], gsem)
            if not isinstance(g, int) or g + 6 <= LAST:
                issue_idx(g + 6, (ig + 6) % NI)

        for g in range(6):
            issue_idx(g, g)
        _zero_fill_2d(rows.at[0], GCH, DP)
        r0 = sid * RPT
        for j in range(RPT // GCH):
            pltpu.sync_copy(rows.at[0], acc.at[pl.ds(r0 + j * GCH, GCH)])
        plsc.subcore_barrier()

        wait_idx(0)
        pltpu.async_copy(y_hbm.at[sv.at[0]], rows.at[0], gsem)
        wait_idx(1)
        pltpu.async_copy(y_hbm.at[sv.at[1]], rows.at[1], gsem)

        for g in range(8):
            step(g, g % NI, g % NR)

        def body(m, carry):
            g = m * NI
            for k in range(NI):
                step(g + k, k, k % NR)
            return carry

        lax.fori_loop(1, 14, body, 0)
        for g in range(112, NC_T):
            step(g, g % NI, g % NR)
        wait_rows((LAST - 1) % NR, ssem)
        wait_rows(LAST % NR, ssem)

        plsc.subcore_barrier()
        NBK = RPT // GCH
        hbm_w = []
        for j in range(NBK):
            s = j % 2
            if j >= 2:
                hbm_w[j - 2].wait()
            pltpu.async_copy(acc.at[pl.ds(r0 + j * GCH, GCH)], rows.at[s], gsem).wait()
            hbm_w.append(pltpu.async_copy(
                rows.at[s], out_hbm.at[cid, pl.ds(r0 + j * GCH, GCH)], ssem))
        hbm_w[NBK - 2].wait()
        hbm_w[NBK - 1].wait()

    return agg_kernel(y, src, dst)


RBLK = 1000


def _tc1_body(deg_ref, x_ref, w_ref, s_ref, y_ref):
    deg = deg_ref[:, 0] + deg_ref[:, 1] + 1.0
    s = lax.rsqrt(deg)[:, None]
    s_ref[...] = s
    y_ref[...] = jnp.dot(x_ref[...] * s, w_ref[...],
                         preferred_element_type=jnp.float32)


def _tc1(degp, x, W1):
    return pl.pallas_call(
        _tc1_body,
        grid=(NN // RBLK,),
        in_specs=[
            pl.BlockSpec((RBLK, NCORE), lambda i: (i, 0)),
            pl.BlockSpec((RBLK, 128), lambda i: (i, 0)),
            pl.BlockSpec((128, 128), lambda i: (0, 0)),
        ],
        out_specs=[
            pl.BlockSpec((RBLK, 1), lambda i: (i, 0)),
            pl.BlockSpec((RBLK, 128), lambda i: (i, 0)),
        ],
        out_shape=[
            jax.ShapeDtypeStruct((NN, 1), jnp.float32),
            jax.ShapeDtypeStruct((NN, 128), jnp.float32),
        ],
    )(degp, x, W1)


def _tc2_body(p_ref, y1_ref, s_ref, b1_ref, w2_ref, y2_ref):
    s = s_ref[...]
    agg = p_ref[0][:, :64] + p_ref[1][:, :64] + y1_ref[...][:, :64]
    h1 = jnp.maximum(agg * s + b1_ref[...], 0.0)
    y2_ref[...] = jnp.dot(h1 * s, w2_ref[...],
                          preferred_element_type=jnp.float32)


def _tc2(p1, y1, s, b1, W2):
    return pl.pallas_call(
        _tc2_body,
        grid=(NN // RBLK,),
        in_specs=[
            pl.BlockSpec((NCORE, RBLK, DP), lambda i: (0, i, 0)),
            pl.BlockSpec((RBLK, DP), lambda i: (i, 0)),
            pl.BlockSpec((RBLK, 1), lambda i: (i, 0)),
            pl.BlockSpec((64,), lambda i: (0,)),
            pl.BlockSpec((64, 128), lambda i: (0, 0)),
        ],
        out_specs=pl.BlockSpec((RBLK, 128), lambda i: (i, 0)),
        out_shape=jax.ShapeDtypeStruct((NN, 128), jnp.float32),
    )(p1, y1, s, b1, W2)


def _tc3_body(p_ref, y2_ref, s_ref, b2_ref, wc1_ref, bc1_ref,
              wc2_ref, bc2_ref, o_ref):
    s = s_ref[...]
    agg = p_ref[0][:, :32] + p_ref[1][:, :32] + y2_ref[...][:, :32]
    h2 = jnp.maximum(agg * s + b2_ref[...], 0.0)
    h3 = jnp.maximum(
        jnp.dot(h2, wc1_ref[...], preferred_element_type=jnp.float32)
        + bc1_ref[...], 0.0)
    o_ref[...] = (jnp.dot(h3, wc2_ref[...], preferred_element_type=jnp.float32)
                  + bc2_ref[...])


def _tc3(p2, y2, s, b2, Wc1, bc1, Wc2, bc2):
    return pl.pallas_call(
        _tc3_body,
        grid=(NN // RBLK,),
        in_specs=[
            pl.BlockSpec((NCORE, RBLK, DP), lambda i: (0, i, 0)),
            pl.BlockSpec((RBLK, DP), lambda i: (i, 0)),
            pl.BlockSpec((RBLK, 1), lambda i: (i, 0)),
            pl.BlockSpec((32,), lambda i: (0,)),
            pl.BlockSpec((32, 16), lambda i: (0, 0)),
            pl.BlockSpec((16,), lambda i: (0,)),
            pl.BlockSpec((16, 10), lambda i: (0, 0)),
            pl.BlockSpec((10,), lambda i: (0,)),
        ],
        out_specs=pl.BlockSpec((RBLK, 10), lambda i: (i, 0)),
        out_shape=jax.ShapeDtypeStruct((NN, 10), jnp.float32),
    )(p2, y2, s, b2, Wc1, bc1, Wc2, bc2)


def kernel(x, edge_index, batch, W1, b1, W2, b2, Wc1, bc1, Wc2, bc2):
    src = edge_index[0]
    dst = edge_index[1]
    W1p = jnp.pad(W1, ((0, 0), (0, DP - W1.shape[1])))
    W2p = jnp.pad(W2, ((0, 0), (0, DP - W2.shape[1])))
    degp = _sc_deg(dst)[:, :NN].T
    s, y1 = _tc1(degp, x, W1p)
    p1 = _sc_agg(y1, src, dst)
    y2 = _tc2(p1, y1, s, b1, W2p)
    p2 = _sc_agg(y2, src, dst)
    return _tc3(p2, y2, s, b2, Wc1, bc1, Wc2, bc2)


# speedup vs baseline: 39.7964x; 1.2176x over previous
import functools

import jax
import jax.numpy as jnp
from jax import lax
from jax.experimental import pallas as pl
from jax.experimental.pallas import tpu as pltpu
from jax.experimental.pallas import tpu_sc as plsc

NN = 10000
EE = 320000
NCORE = 2
NSUB = 16
NW = NCORE * NSUB
EPT = EE // NW
GCH = 80
NPAD = 10240
RPT = NPAD // NSUB
RB = 128
RPT_DEG = RPT
CH_D = 80
NCH_D = 5
NB_D = EPT // (CH_D * NCH_D)


def _mesh():
    return plsc.VectorSubcoreMesh(
        core_axis_name="c", subcore_axis_name="s",
        num_cores=NCORE, num_subcores=NSUB)


def _zero_fill_1d(ref, nwords):
    z = jnp.zeros((16,), jnp.float32)

    def body(i, carry):
        ref[pl.ds(i * 16, 16)] = z
        return carry

    lax.fori_loop(0, nwords // 16, body, 0)


def _zero_fill_2d(ref, rows, cols):
    z = jnp.zeros((16,), jnp.float32)
    c16 = cols // 16

    def body(i, carry):
        r = i // c16
        c = i % c16
        ref[r, pl.ds(c * 16, 16)] = z
        return carry

    lax.fori_loop(0, rows * c16, body, 0)


def _sc_deg(dst):
    @functools.partial(
        pl.kernel,
        out_type=jax.ShapeDtypeStruct((NCORE, NPAD), jnp.float32),
        mesh=_mesh(),
        scratch_types=[
            pltpu.VMEM((NCH_D, CH_D), jnp.int32),
            pltpu.VMEM((CH_D,), jnp.float32),
            pltpu.VMEM((RPT_DEG,), jnp.float32),
            pltpu.VMEM_SHARED((NPAD,), jnp.float32),
            pltpu.SemaphoreType.DMA,
            pltpu.SemaphoreType.DMA,
        ],
    )
    def deg_kernel(dst_hbm, out_hbm, dv, ones_v, zb, acc, isem, ssem):
        cid = lax.axis_index("c")
        sid = lax.axis_index("s")
        wid = sid * NCORE + cid
        one = jnp.ones((16,), jnp.float32)
        for k in range(CH_D // 16):
            ones_v[pl.ds(k * 16, 16)] = one
        _zero_fill_1d(zb, RPT_DEG)
        pltpu.sync_copy(zb, acc.at[pl.ds(sid * RPT_DEG, RPT_DEG)])
        plsc.subcore_barrier()

        base = wid * EPT

        def burst(b, carry):
            off = pl.multiple_of(base + b * CH_D * NCH_D, 8)
            hl = [pltpu.async_copy(dst_hbm.at[pl.ds(off + k * CH_D, CH_D)],
                                   dv.at[k], isem)
                  for k in range(NCH_D)]
            for h in hl:
                h.wait()
            hs = [pltpu.async_copy(ones_v, acc.at[dv.at[k]], ssem, add=True)
                  for k in range(NCH_D)]
            for h in hs:
                h.wait()
            return carry

        lax.fori_loop(0, NB_D, burst, 0)
        plsc.subcore_barrier()
        pltpu.sync_copy(acc.at[pl.ds(sid * RPT_DEG, RPT_DEG)], zb)
        pltpu.sync_copy(zb, out_hbm.at[cid, pl.ds(sid * RPT_DEG, RPT_DEG)])

    return deg_kernel(dst)


DP = 128


def _sc_agg(y, src, dst, D):
    NC_T = EPT // GCH
    NI = 8
    NR = 4

    @functools.partial(
        pl.kernel,
        out_type=jax.ShapeDtypeStruct((NCORE, NPAD, D), jnp.float32),
        mesh=_mesh(),
        compiler_params=pltpu.CompilerParams(use_tc_tiling_on_sc=False),
        scratch_types=[
            pltpu.VMEM((NI, GCH), jnp.int32),
            pltpu.VMEM((NI, GCH), jnp.int32),
            pltpu.VMEM((NR, GCH, D), jnp.float32),
            pltpu.VMEM_SHARED((NPAD, D), jnp.float32),
            pltpu.SemaphoreType.DMA,
            pltpu.SemaphoreType.DMA,
            pltpu.SemaphoreType.DMA,
        ],
    )
    def agg_kernel(y_hbm, src_hbm, dst_hbm, out_hbm,
                   sv, dv, rows, acc, isem, gsem, ssem):
        cid = lax.axis_index("c")
        sid = lax.axis_index("s")
        wid = sid * NCORE + cid
        base = wid * EPT

        def issue_idx(g, ig):
            off = pl.multiple_of(base + g * GCH, 8)
            pltpu.async_copy(src_hbm.at[pl.ds(off, GCH)], sv.at[ig], isem)
            pltpu.async_copy(dst_hbm.at[pl.ds(off, GCH)], dv.at[ig], isem)

        def wait_idx(ig):
            pltpu.make_async_copy(src_hbm.at[pl.ds(0, GCH)], sv.at[ig], isem).wait()
            pltpu.make_async_copy(dst_hbm.at[pl.ds(0, GCH)], dv.at[ig], isem).wait()

        def wait_rows(rg, sem):
            pltpu.make_async_copy(y_hbm.at[pl.ds(0, GCH)], rows.at[rg], sem).wait()

        LAST = NC_T - 1

        def step(g, ig, rg):
            wait_rows(rg, gsem)
            pltpu.async_copy(rows.at[rg], acc.at[dv.at[ig]], ssem, add=True)
            if not isinstance(g, int) or g >= 2:
                wait_rows((rg + 2) % NR, ssem)
            if not isinstance(g, int) or g + 2 <= LAST:
                ig2 = (ig + 2) % NI
                wait_idx(ig2)
                pltpu.async_copy(y_hbm.at[sv.at[ig2]],
                                 rows.at[(rg + 2) % NR], gsem)
            if not isinstance(g, int) or g + 6 <= LAST:
                issue_idx(g + 6, (ig + 6) % NI)

        for g in range(6):
            issue_idx(g, g)
        _zero_fill_2d(rows.at[0], GCH, D)
        r0 = sid * RPT
        for j in range(RPT // GCH):
            pltpu.sync_copy(rows.at[0], acc.at[pl.ds(r0 + j * GCH, GCH)])
        plsc.subcore_barrier()

        wait_idx(0)
        pltpu.async_copy(y_hbm.at[sv.at[0]], rows.at[0], gsem)
        wait_idx(1)
        pltpu.async_copy(y_hbm.at[sv.at[1]], rows.at[1], gsem)

        for g in range(8):
            step(g, g % NI, g % NR)

        def body(m, carry):
            g = m * NI
            for k in range(NI):
                step(g + k, k, k % NR)
            return carry

        lax.fori_loop(1, 14, body, 0)
        for g in range(112, NC_T):
            step(g, g % NI, g % NR)
        wait_rows((LAST - 1) % NR, ssem)
        wait_rows(LAST % NR, ssem)

        plsc.subcore_barrier()
        NBK = RPT // GCH
        hbm_w = []
        for j in range(NBK):
            s = j % 2
            if j >= 2:
                hbm_w[j - 2].wait()
            pltpu.async_copy(acc.at[pl.ds(r0 + j * GCH, GCH)], rows.at[s], gsem).wait()
            hbm_w.append(pltpu.async_copy(
                rows.at[s], out_hbm.at[cid, pl.ds(r0 + j * GCH, GCH)], ssem))
        hbm_w[NBK - 2].wait()
        hbm_w[NBK - 1].wait()

    return agg_kernel(y, src, dst)


RBLK = 1000


def _tc1_body(deg_ref, x_ref, w_ref, s_ref, y_ref):
    deg = deg_ref[:, 0] + deg_ref[:, 1] + 1.0
    s = lax.rsqrt(deg)[:, None]
    s_ref[...] = s
    y_ref[...] = jnp.dot(x_ref[...] * s, w_ref[...],
                         preferred_element_type=jnp.float32)


def _tc1(degp, x, W1):
    return pl.pallas_call(
        _tc1_body,
        grid=(NN // RBLK,),
        in_specs=[
            pl.BlockSpec((RBLK, NCORE), lambda i: (i, 0)),
            pl.BlockSpec((RBLK, 128), lambda i: (i, 0)),
            pl.BlockSpec((128, 64), lambda i: (0, 0)),
        ],
        out_specs=[
            pl.BlockSpec((RBLK, 1), lambda i: (i, 0)),
            pl.BlockSpec((RBLK, 64), lambda i: (i, 0)),
        ],
        out_shape=[
            jax.ShapeDtypeStruct((NN, 1), jnp.float32),
            jax.ShapeDtypeStruct((NN, 64), jnp.float32),
        ],
    )(degp, x, W1)


def _tc2_body(p_ref, y1_ref, s_ref, b1_ref, w2_ref, y2_ref):
    s = s_ref[...]
    agg = p_ref[0] + p_ref[1] + y1_ref[...]
    h1 = jnp.maximum(agg * s + b1_ref[...], 0.0)
    y2_ref[...] = jnp.dot(h1 * s, w2_ref[...],
                          preferred_element_type=jnp.float32)


def _tc2(p1, y1, s, b1, W2):
    return pl.pallas_call(
        _tc2_body,
        grid=(NN // RBLK,),
        in_specs=[
            pl.BlockSpec((NCORE, RBLK, 64), lambda i: (0, i, 0)),
            pl.BlockSpec((RBLK, 64), lambda i: (i, 0)),
            pl.BlockSpec((RBLK, 1), lambda i: (i, 0)),
            pl.BlockSpec((64,), lambda i: (0,)),
            pl.BlockSpec((64, 32), lambda i: (0, 0)),
        ],
        out_specs=pl.BlockSpec((RBLK, 32), lambda i: (i, 0)),
        out_shape=jax.ShapeDtypeStruct((NN, 32), jnp.float32),
    )(p1, y1, s, b1, W2)


def _tc3_body(p_ref, y2_ref, s_ref, b2_ref, wc1_ref, bc1_ref,
              wc2_ref, bc2_ref, o_ref):
    s = s_ref[...]
    agg = p_ref[0] + p_ref[1] + y2_ref[...]
    h2 = jnp.maximum(agg * s + b2_ref[...], 0.0)
    h3 = jnp.maximum(
        jnp.dot(h2, wc1_ref[...], preferred_element_type=jnp.float32)
        + bc1_ref[...], 0.0)
    o_ref[...] = (jnp.dot(h3, wc2_ref[...], preferred_element_type=jnp.float32)
                  + bc2_ref[...])


def _tc3(p2, y2, s, b2, Wc1, bc1, Wc2, bc2):
    return pl.pallas_call(
        _tc3_body,
        grid=(NN // RBLK,),
        in_specs=[
            pl.BlockSpec((NCORE, RBLK, 32), lambda i: (0, i, 0)),
            pl.BlockSpec((RBLK, 32), lambda i: (i, 0)),
            pl.BlockSpec((RBLK, 1), lambda i: (i, 0)),
            pl.BlockSpec((32,), lambda i: (0,)),
            pl.BlockSpec((32, 16), lambda i: (0, 0)),
            pl.BlockSpec((16,), lambda i: (0,)),
            pl.BlockSpec((16, 10), lambda i: (0, 0)),
            pl.BlockSpec((10,), lambda i: (0,)),
        ],
        out_specs=pl.BlockSpec((RBLK, 10), lambda i: (i, 0)),
        out_shape=jax.ShapeDtypeStruct((NN, 10), jnp.float32),
    )(p2, y2, s, b2, Wc1, bc1, Wc2, bc2)


def kernel(x, edge_index, batch, W1, b1, W2, b2, Wc1, bc1, Wc2, bc2):
    src = edge_index[0]
    dst = edge_index[1]
    degp = _sc_deg(dst)[:, :NN].T
    s, y1 = _tc1(degp, x, W1)
    p1 = _sc_agg(y1, src, dst, 64)
    y2 = _tc2(p1, y1, s, b1, W2)
    p2 = _sc_agg(y2, src, dst, 32)
    return _tc3(p2, y2, s, b2, Wc1, bc1, Wc2, bc2)


# speedup vs baseline: 43.6045x; 1.0957x over previous
import functools

import jax
import jax.numpy as jnp
from jax import lax
from jax.experimental import pallas as pl
from jax.experimental.pallas import tpu as pltpu
from jax.experimental.pallas import tpu_sc as plsc

NN = 10000
EE = 320000
NCORE = 2
NSUB = 16
NW = NCORE * NSUB
EPT = EE // NW
GCH = 80
NPAD = 10240
RPT = NPAD // NSUB
RB = 128
RPT_DEG = RPT
CH_D = 80
NCH_D = 5
NB_D = EPT // (CH_D * NCH_D)


def _mesh():
    return plsc.VectorSubcoreMesh(
        core_axis_name="c", subcore_axis_name="s",
        num_cores=NCORE, num_subcores=NSUB)


def _zero_fill_1d(ref, nwords):
    z = jnp.zeros((16,), jnp.float32)

    def body(i, carry):
        ref[pl.ds(i * 16, 16)] = z
        return carry

    lax.fori_loop(0, nwords // 16, body, 0)


def _zero_fill_2d(ref, rows, cols):
    z = jnp.zeros((16,), jnp.float32)
    c16 = cols // 16

    def body(i, carry):
        r = i // c16
        c = i % c16
        ref[r, pl.ds(c * 16, 16)] = z
        return carry

    lax.fori_loop(0, rows * c16, body, 0)


def _sc_deg(dst):
    @functools.partial(
        pl.kernel,
        out_type=jax.ShapeDtypeStruct((NCORE, NPAD), jnp.float32),
        mesh=_mesh(),
        scratch_types=[
            pltpu.VMEM((NCH_D, CH_D), jnp.int32),
            pltpu.VMEM((CH_D,), jnp.float32),
            pltpu.VMEM((RPT_DEG,), jnp.float32),
            pltpu.VMEM_SHARED((NPAD,), jnp.float32),
            pltpu.SemaphoreType.DMA,
            pltpu.SemaphoreType.DMA,
        ],
    )
    def deg_kernel(dst_hbm, out_hbm, dv, ones_v, zb, acc, isem, ssem):
        cid = lax.axis_index("c")
        sid = lax.axis_index("s")
        wid = sid * NCORE + cid
        one = jnp.ones((16,), jnp.float32)
        for k in range(CH_D // 16):
            ones_v[pl.ds(k * 16, 16)] = one
        _zero_fill_1d(zb, RPT_DEG)
        pltpu.sync_copy(zb, acc.at[pl.ds(sid * RPT_DEG, RPT_DEG)])
        plsc.subcore_barrier()

        base = wid * EPT

        def burst(b, carry):
            off = pl.multiple_of(base + b * CH_D * NCH_D, 8)
            hl = [pltpu.async_copy(dst_hbm.at[pl.ds(off + k * CH_D, CH_D)],
                                   dv.at[k], isem)
                  for k in range(NCH_D)]
            for h in hl:
                h.wait()
            hs = [pltpu.async_copy(ones_v, acc.at[dv.at[k]], ssem, add=True)
                  for k in range(NCH_D)]
            for h in hs:
                h.wait()
            return carry

        lax.fori_loop(0, NB_D, burst, 0)
        plsc.subcore_barrier()
        pltpu.sync_copy(acc.at[pl.ds(sid * RPT_DEG, RPT_DEG)], zb)
        pltpu.sync_copy(zb, out_hbm.at[cid, pl.ds(sid * RPT_DEG, RPT_DEG)])

    return deg_kernel(dst)


DP = 128


def _sc_agg(y, src, dst, D):
    NC_T = EPT // GCH
    NI = 8
    NR = 4

    @functools.partial(
        pl.kernel,
        out_type=jax.ShapeDtypeStruct((NCORE, NPAD, D), jnp.float32),
        mesh=_mesh(),
        compiler_params=pltpu.CompilerParams(use_tc_tiling_on_sc=False),
        scratch_types=[
            pltpu.VMEM((NI, GCH), jnp.int32),
            pltpu.VMEM((NI, GCH), jnp.int32),
            pltpu.VMEM((NR, GCH, D), jnp.float32),
            pltpu.VMEM_SHARED((NPAD, D), jnp.float32),
            pltpu.SemaphoreType.DMA,
            pltpu.SemaphoreType.DMA,
            pltpu.SemaphoreType.DMA,
        ],
    )
    def agg_kernel(y_hbm, src_hbm, dst_hbm, out_hbm,
                   sv, dv, rows, acc, isem, gsem, ssem):
        cid = lax.axis_index("c")
        sid = lax.axis_index("s")
        wid = sid * NCORE + cid
        base = wid * EPT

        def issue_idx(g, ig):
            off = pl.multiple_of(base + g * GCH, 8)
            pltpu.async_copy(src_hbm.at[pl.ds(off, GCH)], sv.at[ig], isem)
            pltpu.async_copy(dst_hbm.at[pl.ds(off, GCH)], dv.at[ig], isem)

        def wait_idx(ig):
            pltpu.make_async_copy(src_hbm.at[pl.ds(0, GCH)], sv.at[ig], isem).wait()
            pltpu.make_async_copy(dst_hbm.at[pl.ds(0, GCH)], dv.at[ig], isem).wait()

        def wait_rows(rg, sem):
            pltpu.make_async_copy(y_hbm.at[pl.ds(0, GCH)], rows.at[rg], sem).wait()

        LAST = NC_T - 1

        def step(g, ig, rg):
            wait_rows(rg, gsem)
            pltpu.async_copy(rows.at[rg], acc.at[dv.at[ig]], ssem, add=True)
            if not isinstance(g, int) or g >= 2:
                wait_rows((rg + 2) % NR, ssem)
            if not isinstance(g, int) or g + 2 <= LAST:
                ig2 = (ig + 2) % NI
                wait_idx(ig2)
                pltpu.async_copy(y_hbm.at[sv.at[ig2]],
                                 rows.at[(rg + 2) % NR], gsem)
            if not isinstance(g, int) or g + 6 <= LAST:
                issue_idx(g + 6, (ig + 6) % NI)

        for g in range(6):
            issue_idx(g, g)
        _zero_fill_2d(rows.at[0], GCH, D)
        r0 = sid * RPT
        for j in range(RPT // GCH):
            pltpu.sync_copy(rows.at[0], acc.at[pl.ds(r0 + j * GCH, GCH)])
        plsc.subcore_barrier()

        wait_idx(0)
        pltpu.async_copy(y_hbm.at[sv.at[0]], rows.at[0], gsem)
        wait_idx(1)
        pltpu.async_copy(y_hbm.at[sv.at[1]], rows.at[1], gsem)

        for g in range(8):
            step(g, g % NI, g % NR)

        def body(m, carry):
            g = m * NI
            for k in range(NI):
                step(g + k, k, k % NR)
            return carry

        lax.fori_loop(1, 14, body, 0)
        for g in range(112, NC_T):
            step(g, g % NI, g % NR)
        wait_rows((LAST - 1) % NR, ssem)
        wait_rows(LAST % NR, ssem)

        plsc.subcore_barrier()
        NBK = RPT // GCH
        hbm_w = []
        for j in range(NBK):
            s = j % 2
            if j >= 2:
                hbm_w[j - 2].wait()
            pltpu.async_copy(acc.at[pl.ds(r0 + j * GCH, GCH)], rows.at[s], gsem).wait()
            hbm_w.append(pltpu.async_copy(
                rows.at[s], out_hbm.at[cid, pl.ds(r0 + j * GCH, GCH)], ssem))
        hbm_w[NBK - 2].wait()
        hbm_w[NBK - 1].wait()

    return agg_kernel(y, src, dst)


RBLK = 2048
EPAD = 327680
EBLK = EPAD // 5


def _tc0_body(ei_ref, x_ref, w_ref, src_ref, dst_ref, xw_ref):
    src_ref[...] = ei_ref[0, :]
    dst_ref[...] = ei_ref[1, :]
    xw_ref[...] = jnp.dot(x_ref[...], w_ref[...],
                          preferred_element_type=jnp.float32)


def _tc0(edge_index, x, W1):
    return pl.pallas_call(
        _tc0_body,
        grid=(5,),
        in_specs=[
            pl.BlockSpec((2, EBLK), lambda i: (0, i)),
            pl.BlockSpec((RBLK, 128), lambda i: (i, 0)),
            pl.BlockSpec((128, 64), lambda i: (0, 0)),
        ],
        out_specs=[
            pl.BlockSpec((EBLK,), lambda i: (i,)),
            pl.BlockSpec((EBLK,), lambda i: (i,)),
            pl.BlockSpec((RBLK, 64), lambda i: (i, 0)),
        ],
        out_shape=[
            jax.ShapeDtypeStruct((EPAD,), jnp.int32),
            jax.ShapeDtypeStruct((EPAD,), jnp.int32),
            jax.ShapeDtypeStruct((NN, 64), jnp.float32),
        ],
    )(edge_index, x, W1)


def _tc1_body(deg_ref, xw_ref, s_ref, y_ref):
    deg = deg_ref[0, :] + deg_ref[1, :] + 1.0
    s = lax.rsqrt(deg)[:, None]
    s_ref[...] = s
    y_ref[...] = xw_ref[...] * s


def _tc1(degp, xw):
    return pl.pallas_call(
        _tc1_body,
        grid=(5,),
        in_specs=[
            pl.BlockSpec((NCORE, RBLK), lambda i: (0, i)),
            pl.BlockSpec((RBLK, 64), lambda i: (i, 0)),
        ],
        out_specs=[
            pl.BlockSpec((RBLK, 1), lambda i: (i, 0)),
            pl.BlockSpec((RBLK, 64), lambda i: (i, 0)),
        ],
        out_shape=[
            jax.ShapeDtypeStruct((NN, 1), jnp.float32),
            jax.ShapeDtypeStruct((NN, 64), jnp.float32),
        ],
    )(degp, xw)


def _tc2_body(p_ref, y1_ref, s_ref, b1_ref, w2_ref, y2_ref):
    s = s_ref[...]
    agg = p_ref[0] + p_ref[1] + y1_ref[...]
    h1 = jnp.maximum(agg * s + b1_ref[...], 0.0)
    y2_ref[...] = jnp.dot(h1 * s, w2_ref[...],
                          preferred_element_type=jnp.float32)


def _tc2(p1, y1, s, b1, W2):
    return pl.pallas_call(
        _tc2_body,
        grid=(5,),
        in_specs=[
            pl.BlockSpec((NCORE, RBLK, 64), lambda i: (0, i, 0)),
            pl.BlockSpec((RBLK, 64), lambda i: (i, 0)),
            pl.BlockSpec((RBLK, 1), lambda i: (i, 0)),
            pl.BlockSpec((64,), lambda i: (0,)),
            pl.BlockSpec((64, 32), lambda i: (0, 0)),
        ],
        out_specs=pl.BlockSpec((RBLK, 32), lambda i: (i, 0)),
        out_shape=jax.ShapeDtypeStruct((NN, 32), jnp.float32),
    )(p1, y1, s, b1, W2)


def _tc3_body(p_ref, y2_ref, s_ref, b2_ref, wc1_ref, bc1_ref,
              wc2_ref, bc2_ref, o_ref):
    s = s_ref[...]
    agg = p_ref[0] + p_ref[1] + y2_ref[...]
    h2 = jnp.maximum(agg * s + b2_ref[...], 0.0)
    h3 = jnp.maximum(
        jnp.dot(h2, wc1_ref[...], preferred_element_type=jnp.float32)
        + bc1_ref[...], 0.0)
    o_ref[...] = (jnp.dot(h3, wc2_ref[...], preferred_element_type=jnp.float32)
                  + bc2_ref[...])


def _tc3(p2, y2, s, b2, Wc1, bc1, Wc2, bc2):
    return pl.pallas_call(
        _tc3_body,
        grid=(5,),
        in_specs=[
            pl.BlockSpec((NCORE, RBLK, 32), lambda i: (0, i, 0)),
            pl.BlockSpec((RBLK, 32), lambda i: (i, 0)),
            pl.BlockSpec((RBLK, 1), lambda i: (i, 0)),
            pl.BlockSpec((32,), lambda i: (0,)),
            pl.BlockSpec((32, 16), lambda i: (0, 0)),
            pl.BlockSpec((16,), lambda i: (0,)),
            pl.BlockSpec((16, 10), lambda i: (0, 0)),
            pl.BlockSpec((10,), lambda i: (0,)),
        ],
        out_specs=pl.BlockSpec((RBLK, 10), lambda i: (i, 0)),
        out_shape=jax.ShapeDtypeStruct((NN, 10), jnp.float32),
    )(p2, y2, s, b2, Wc1, bc1, Wc2, bc2)


def kernel(x, edge_index, batch, W1, b1, W2, b2, Wc1, bc1, Wc2, bc2):
    src, dst, xw = _tc0(edge_index, x, W1)
    degp = _sc_deg(dst)
    s, y1 = _tc1(degp, xw)
    p1 = _sc_agg(y1, src, dst, 64)
    y2 = _tc2(p1, y1, s, b1, W2)
    p2 = _sc_agg(y2, src, dst, 32)
    return _tc3(p2, y2, s, b2, Wc1, bc1, Wc2, bc2)


# speedup vs baseline: 49.1801x; 1.1279x over previous
import functools

import jax
import jax.numpy as jnp
from jax import lax
from jax.experimental import pallas as pl
from jax.experimental.pallas import tpu as pltpu
from jax.experimental.pallas import tpu_sc as plsc

NN = 10000
EE = 320000
NCORE = 2
NSUB = 16
NW = NCORE * NSUB
EPT = EE // NW
GCH = 80
NPAD = 10240
RPT = NPAD // NSUB
RB = 128
RPT_DEG = RPT
CH_D = 80
NCH_D = 5
NB_D = EPT // (CH_D * NCH_D)


def _mesh():
    return plsc.VectorSubcoreMesh(
        core_axis_name="c", subcore_axis_name="s",
        num_cores=NCORE, num_subcores=NSUB)


def _zero_fill_1d(ref, nwords):
    z = jnp.zeros((16,), jnp.float32)

    def body(i, carry):
        ref[pl.ds(i * 16, 16)] = z
        return carry

    lax.fori_loop(0, nwords // 16, body, 0)


def _zero_fill_2d(ref, rows, cols):
    z = jnp.zeros((16,), jnp.float32)
    c16 = cols // 16

    def body(i, carry):
        r = i // c16
        c = i % c16
        ref[r, pl.ds(c * 16, 16)] = z
        return carry

    lax.fori_loop(0, rows * c16, body, 0)


def _sc_deg(dst):
    NID = 8
    NC_D = EPT // CH_D

    @functools.partial(
        pl.kernel,
        out_type=jax.ShapeDtypeStruct((NCORE, NPAD), jnp.float32),
        mesh=_mesh(),
        scratch_types=[
            pltpu.VMEM((NID, CH_D), jnp.int32),
            pltpu.VMEM((CH_D,), jnp.float32),
            pltpu.VMEM((RPT_DEG,), jnp.float32),
            pltpu.VMEM_SHARED((NPAD,), jnp.float32),
            pltpu.SemaphoreType.DMA,
            pltpu.SemaphoreType.DMA,
        ],
    )
    def deg_kernel(dst_hbm, out_hbm, dv, ones_v, zb, acc, isem, ssem):
        cid = lax.axis_index("c")
        sid = lax.axis_index("s")
        wid = sid * NCORE + cid
        base = wid * EPT
        one = jnp.ones((16,), jnp.float32)
        for k in range(CH_D // 16):
            ones_v[pl.ds(k * 16, 16)] = one

        def issue_idx(g, ig):
            off = pl.multiple_of(base + g * CH_D, 8)
            pltpu.async_copy(dst_hbm.at[pl.ds(off, CH_D)], dv.at[ig], isem)

        def wait_one(ig, sem):
            pltpu.make_async_copy(dst_hbm.at[pl.ds(0, CH_D)], dv.at[ig], sem).wait()

        def step(g, ig):
            wait_one(ig, isem)
            pltpu.async_copy(ones_v, acc.at[dv.at[ig]], ssem, add=True)
            if not isinstance(g, int) or g >= 4:
                wait_one((ig + 4) % NID, ssem)
            if not isinstance(g, int) or g + 4 <= NC_D - 1:
                issue_idx(g + 4, (ig + 4) % NID)

        for g in range(4):
            issue_idx(g, g)
        _zero_fill_1d(zb, RPT_DEG)
        pltpu.sync_copy(zb, acc.at[pl.ds(sid * RPT_DEG, RPT_DEG)])
        plsc.subcore_barrier()

        for g in range(8):
            step(g, g)

        def body(m, carry):
            for k in range(NID):
                step(m * NID + k, k)
            return carry

        lax.fori_loop(1, 15, body, 0)
        for g in range(120, 125):
            step(g, g % NID)
        for g in range(121, 125):
            wait_one(g % NID, ssem)

        plsc.subcore_barrier()
        pltpu.sync_copy(acc.at[pl.ds(sid * RPT_DEG, RPT_DEG)], zb)
        pltpu.sync_copy(zb, out_hbm.at[cid, pl.ds(sid * RPT_DEG, RPT_DEG)])

    return deg_kernel(dst)


DP = 128


def _sc_agg(y, src, dst, D):
    NC_T = EPT // GCH
    NI = 16
    NR = 8

    @functools.partial(
        pl.kernel,
        out_type=jax.ShapeDtypeStruct((NCORE, NPAD, D), jnp.float32),
        mesh=_mesh(),
        compiler_params=pltpu.CompilerParams(use_tc_tiling_on_sc=False),
        scratch_types=[
            pltpu.VMEM((NI, GCH), jnp.int32),
            pltpu.VMEM((NI, GCH), jnp.int32),
            pltpu.VMEM((NR, GCH, D), jnp.float32),
            pltpu.VMEM_SHARED((NPAD, D), jnp.float32),
            pltpu.SemaphoreType.DMA,
            pltpu.SemaphoreType.DMA,
            pltpu.SemaphoreType.DMA,
        ],
    )
    def agg_kernel(y_hbm, src_hbm, dst_hbm, out_hbm,
                   sv, dv, rows, acc, isem, gsem, ssem):
        cid = lax.axis_index("c")
        sid = lax.axis_index("s")
        wid = sid * NCORE + cid
        base = wid * EPT

        def issue_idx(g, ig):
            off = pl.multiple_of(base + g * GCH, 8)
            pltpu.async_copy(src_hbm.at[pl.ds(off, GCH)], sv.at[ig], isem)
            pltpu.async_copy(dst_hbm.at[pl.ds(off, GCH)], dv.at[ig], isem)

        def wait_idx(ig):
            pltpu.make_async_copy(src_hbm.at[pl.ds(0, GCH)], sv.at[ig], isem).wait()
            pltpu.make_async_copy(dst_hbm.at[pl.ds(0, GCH)], dv.at[ig], isem).wait()

        def wait_rows(rg, sem):
            pltpu.make_async_copy(y_hbm.at[pl.ds(0, GCH)], rows.at[rg], sem).wait()

        LAST = NC_T - 1

        def step(g, ig, rg):
            wait_rows(rg, gsem)
            pltpu.async_copy(rows.at[rg], acc.at[dv.at[ig]], ssem, add=True)
            if not isinstance(g, int) or g >= 4:
                wait_rows((rg + 4) % NR, ssem)
            if not isinstance(g, int) or g + 3 <= LAST:
                ig3 = (ig + 3) % NI
                wait_idx(ig3)
                pltpu.async_copy(y_hbm.at[sv.at[ig3]],
                                 rows.at[(rg + 3) % NR], gsem)
            if not isinstance(g, int) or g + 6 <= LAST:
                issue_idx(g + 6, (ig + 6) % NI)

        for g in range(6):
            issue_idx(g, g)
        _zero_fill_2d(rows.at[0], GCH, D)
        r0 = sid * RPT
        for j in range(RPT // GCH):
            pltpu.sync_copy(rows.at[0], acc.at[pl.ds(r0 + j * GCH, GCH)])
        plsc.subcore_barrier()

        for g in range(3):
            wait_idx(g)
            pltpu.async_copy(y_hbm.at[sv.at[g]], rows.at[g], gsem)

        for g in range(16):
            step(g, g % NI, g % NR)

        def body(m, carry):
            g = m * NI
            for k in range(NI):
                step(g + k, k, k % NR)
            return carry

        lax.fori_loop(1, 7, body, 0)
        for g in range(112, NC_T):
            step(g, g % NI, g % NR)
        for g in range(121, 125):
            wait_rows(g % NR, ssem)

        plsc.subcore_barrier()
        NBK = RPT // GCH
        hbm_w = []
        for j in range(NBK):
            s = j % 2
            if j >= 2:
                hbm_w[j - 2].wait()
            pltpu.async_copy(acc.at[pl.ds(r0 + j * GCH, GCH)], rows.at[s], gsem).wait()
            hbm_w.append(pltpu.async_copy(
                rows.at[s], out_hbm.at[cid, pl.ds(r0 + j * GCH, GCH)], ssem))
        hbm_w[NBK - 2].wait()
        hbm_w[NBK - 1].wait()

    return agg_kernel(y, src, dst)


RBLK = 2048
EPAD = 327680
EBLK = EPAD // 5


def _tc0_body(ei_ref, x_ref, w_ref, src_ref, dst_ref, xw_ref):
    src_ref[...] = ei_ref[0, :]
    dst_ref[...] = ei_ref[1, :]
    xw_ref[...] = jnp.dot(x_ref[...], w_ref[...],
                          preferred_element_type=jnp.float32)


def _tc0(edge_index, x, W1):
    return pl.pallas_call(
        _tc0_body,
        grid=(5,),
        in_specs=[
            pl.BlockSpec((2, EBLK), lambda i: (0, i)),
            pl.BlockSpec((RBLK, 128), lambda i: (i, 0)),
            pl.BlockSpec((128, 64), lambda i: (0, 0)),
        ],
        out_specs=[
            pl.BlockSpec((EBLK,), lambda i: (i,)),
            pl.BlockSpec((EBLK,), lambda i: (i,)),
            pl.BlockSpec((RBLK, 64), lambda i: (i, 0)),
        ],
        out_shape=[
            jax.ShapeDtypeStruct((EPAD,), jnp.int32),
            jax.ShapeDtypeStruct((EPAD,), jnp.int32),
            jax.ShapeDtypeStruct((NN, 64), jnp.float32),
        ],
    )(edge_index, x, W1)


def _tc1_body(deg_ref, xw_ref, s_ref, y_ref):
    deg = deg_ref[0, :] + deg_ref[1, :] + 1.0
    s = lax.rsqrt(deg)[:, None]
    s_ref[...] = s
    y_ref[...] = xw_ref[...] * s


def _tc1(degp, xw):
    return pl.pallas_call(
        _tc1_body,
        grid=(5,),
        in_specs=[
            pl.BlockSpec((NCORE, RBLK), lambda i: (0, i)),
            pl.BlockSpec((RBLK, 64), lambda i: (i, 0)),
        ],
        out_specs=[
            pl.BlockSpec((RBLK, 1), lambda i: (i, 0)),
            pl.BlockSpec((RBLK, 64), lambda i: (i, 0)),
        ],
        out_shape=[
            jax.ShapeDtypeStruct((NN, 1), jnp.float32),
            jax.ShapeDtypeStruct((NN, 64), jnp.float32),
        ],
    )(degp, xw)


def _tc2_body(p_ref, y1_ref, s_ref, b1_ref, w2_ref, y2_ref):
    s = s_ref[...]
    agg = p_ref[0] + p_ref[1] + y1_ref[...]
    h1 = jnp.maximum(agg * s + b1_ref[...], 0.0)
    y2_ref[...] = jnp.dot(h1 * s, w2_ref[...],
                          preferred_element_type=jnp.float32)


def _tc2(p1, y1, s, b1, W2):
    return pl.pallas_call(
        _tc2_body,
        grid=(5,),
        in_specs=[
            pl.BlockSpec((NCORE, RBLK, 64), lambda i: (0, i, 0)),
            pl.BlockSpec((RBLK, 64), lambda i: (i, 0)),
            pl.BlockSpec((RBLK, 1), lambda i: (i, 0)),
            pl.BlockSpec((64,), lambda i: (0,)),
            pl.BlockSpec((64, 32), lambda i: (0, 0)),
        ],
        out_specs=pl.BlockSpec((RBLK, 32), lambda i: (i, 0)),
        out_shape=jax.ShapeDtypeStruct((NN, 32), jnp.float32),
    )(p1, y1, s, b1, W2)


def _tc3_body(p_ref, y2_ref, s_ref, b2_ref, wc1_ref, bc1_ref,
              wc2_ref, bc2_ref, o_ref):
    s = s_ref[...]
    agg = p_ref[0] + p_ref[1] + y2_ref[...]
    h2 = jnp.maximum(agg * s + b2_ref[...], 0.0)
    h3 = jnp.maximum(
        jnp.dot(h2, wc1_ref[...], preferred_element_type=jnp.float32)
        + bc1_ref[...], 0.0)
    o_ref[...] = (jnp.dot(h3, wc2_ref[...], preferred_element_type=jnp.float32)
                  + bc2_ref[...])


def _tc3(p2, y2, s, b2, Wc1, bc1, Wc2, bc2):
    return pl.pallas_call(
        _tc3_body,
        grid=(5,),
        in_specs=[
            pl.BlockSpec((NCORE, RBLK, 32), lambda i: (0, i, 0)),
            pl.BlockSpec((RBLK, 32), lambda i: (i, 0)),
            pl.BlockSpec((RBLK, 1), lambda i: (i, 0)),
            pl.BlockSpec((32,), lambda i: (0,)),
            pl.BlockSpec((32, 16), lambda i: (0, 0)),
            pl.BlockSpec((16,), lambda i: (0,)),
            pl.BlockSpec((16, 10), lambda i: (0, 0)),
            pl.BlockSpec((10,), lambda i: (0,)),
        ],
        out_specs=pl.BlockSpec((RBLK, 10), lambda i: (i, 0)),
        out_shape=jax.ShapeDtypeStruct((NN, 10), jnp.float32),
    )(p2, y2, s, b2, Wc1, bc1, Wc2, bc2)


def kernel(x, edge_index, batch, W1, b1, W2, b2, Wc1, bc1, Wc2, bc2):
    src, dst, xw = _tc0(edge_index, x, W1)
    degp = _sc_deg(dst)
    s, y1 = _tc1(degp, xw)
    p1 = _sc_agg(y1, src, dst, 64)
    y2 = _tc2(p1, y1, s, b1, W2)
    p2 = _sc_agg(y2, src, dst, 32)
    return _tc3(p2, y2, s, b2, Wc1, bc1, Wc2, bc2)


# speedup vs baseline: 53.9168x; 1.0963x over previous
import functools

import jax
import jax.numpy as jnp
from jax import lax
from jax.experimental import pallas as pl
from jax.experimental.pallas import tpu as pltpu
from jax.experimental.pallas import tpu_sc as plsc

NN = 10000
EE = 320000
NCORE = 2
NSUB = 16
NW = NCORE * NSUB
EPT = EE // NW
GCH = 80
NPAD = 10240
RPT = NPAD // NSUB
RB = 128
RPT_DEG = RPT
CH_D = 80
NCH_D = 5
NB_D = EPT // (CH_D * NCH_D)


def _mesh():
    return plsc.VectorSubcoreMesh(
        core_axis_name="c", subcore_axis_name="s",
        num_cores=NCORE, num_subcores=NSUB)


def _zero_fill_1d(ref, nwords):
    z = jnp.zeros((16,), jnp.float32)

    def body(i, carry):
        ref[pl.ds(i * 16, 16)] = z
        return carry

    lax.fori_loop(0, nwords // 16, body, 0)


def _zero_fill_2d(ref, rows, cols):
    z = jnp.zeros((16,), jnp.float32)
    c16 = cols // 16

    def body(i, carry):
        r = i // c16
        c = i % c16
        ref[r, pl.ds(c * 16, 16)] = z
        return carry

    lax.fori_loop(0, rows * c16, body, 0)


def _sc_deg(dst):
    NID = 8
    NC_D = EPT // CH_D

    @functools.partial(
        pl.kernel,
        out_type=jax.ShapeDtypeStruct((NCORE, NPAD), jnp.float32),
        mesh=_mesh(),
        scratch_types=[
            pltpu.VMEM((NID, CH_D), jnp.int32),
            pltpu.VMEM((CH_D,), jnp.float32),
            pltpu.VMEM((RPT_DEG,), jnp.float32),
            pltpu.VMEM_SHARED((NPAD,), jnp.float32),
            pltpu.SemaphoreType.DMA,
            pltpu.SemaphoreType.DMA,
        ],
    )
    def deg_kernel(dst_hbm, out_hbm, dv, ones_v, zb, acc, isem, ssem):
        cid = lax.axis_index("c")
        sid = lax.axis_index("s")
        wid = sid * NCORE + cid
        base = wid * EPT
        one = jnp.ones((16,), jnp.float32)
        for k in range(CH_D // 16):
            ones_v[pl.ds(k * 16, 16)] = one

        def issue_idx(g, ig):
            off = pl.multiple_of(base + g * CH_D, 8)
            pltpu.async_copy(dst_hbm.at[pl.ds(off, CH_D)], dv.at[ig], isem)

        def wait_one(ig, sem):
            pltpu.make_async_copy(dst_hbm.at[pl.ds(0, CH_D)], dv.at[ig], sem).wait()

        def step(g, ig):
            wait_one(ig, isem)
            pltpu.async_copy(ones_v, acc.at[dv.at[ig]], ssem, add=True)
            if not isinstance(g, int) or g >= 4:
                wait_one((ig + 4) % NID, ssem)
            if not isinstance(g, int) or g + 4 <= NC_D - 1:
                issue_idx(g + 4, (ig + 4) % NID)

        for g in range(4):
            issue_idx(g, g)
        _zero_fill_1d(zb, RPT_DEG)
        pltpu.sync_copy(zb, acc.at[pl.ds(sid * RPT_DEG, RPT_DEG)])
        plsc.subcore_barrier()

        for g in range(8):
            step(g, g)

        def body(m, carry):
            for k in range(NID):
                step(m * NID + k, k)
            return carry

        lax.fori_loop(1, 15, body, 0)
        for g in range(120, 125):
            step(g, g % NID)
        for g in range(121, 125):
            wait_one(g % NID, ssem)

        plsc.subcore_barrier()
        pltpu.sync_copy(acc.at[pl.ds(sid * RPT_DEG, RPT_DEG)], zb)
        pltpu.sync_copy(zb, out_hbm.at[cid, pl.ds(sid * RPT_DEG, RPT_DEG)])

    return deg_kernel(dst)


DP = 128


def _sc_agg(y, src, dst, D):
    NC_T = EPT // GCH
    NI = 16
    NR = 8

    @functools.partial(
        pl.kernel,
        out_type=jax.ShapeDtypeStruct((NPAD, 2 * DP if False else 128), jnp.float32),
        mesh=_mesh(),
        compiler_params=pltpu.CompilerParams(use_tc_tiling_on_sc=False),
        scratch_types=[
            pltpu.VMEM((NI, GCH), jnp.int32),
            pltpu.VMEM((NI, GCH), jnp.int32),
            pltpu.VMEM((NR, GCH, D), jnp.float32),
            pltpu.VMEM_SHARED((NPAD, D), jnp.float32),
            pltpu.SemaphoreType.DMA,
            pltpu.SemaphoreType.DMA,
            pltpu.SemaphoreType.DMA,
        ],
    )
    def agg_kernel(y_hbm, src_hbm, dst_hbm, out_hbm,
                   sv, dv, rows, acc, isem, gsem, ssem):
        cid = lax.axis_index("c")
        sid = lax.axis_index("s")
        wid = sid * NCORE + cid
        base = wid * EPT

        def issue_idx(g, ig):
            off = pl.multiple_of(base + g * GCH, 8)
            pltpu.async_copy(src_hbm.at[pl.ds(off, GCH)], sv.at[ig], isem)
            pltpu.async_copy(dst_hbm.at[pl.ds(off, GCH)], dv.at[ig], isem)

        def wait_idx(ig):
            pltpu.make_async_copy(src_hbm.at[pl.ds(0, GCH)], sv.at[ig], isem).wait()
            pltpu.make_async_copy(dst_hbm.at[pl.ds(0, GCH)], dv.at[ig], isem).wait()

        def wait_rows(rg, sem):
            pltpu.make_async_copy(y_hbm.at[pl.ds(0, GCH)], rows.at[rg], sem).wait()

        LAST = NC_T - 1

        def step(g, ig, rg):
            wait_rows(rg, gsem)
            pltpu.async_copy(rows.at[rg], acc.at[dv.at[ig]], ssem, add=True)
            if not isinstance(g, int) or g >= 4:
                wait_rows((rg + 4) % NR, ssem)
            if not isinstance(g, int) or g + 3 <= LAST:
                ig3 = (ig + 3) % NI
                wait_idx(ig3)
                pltpu.async_copy(y_hbm.at[sv.at[ig3]],
                                 rows.at[(rg + 3) % NR], gsem)
            if not isinstance(g, int) or g + 6 <= LAST:
                issue_idx(g + 6, (ig + 6) % NI)

        for g in range(6):
            issue_idx(g, g)
        _zero_fill_2d(rows.at[0], GCH, D)
        r0 = sid * RPT
        for j in range(RPT // GCH):
            pltpu.sync_copy(rows.at[0], acc.at[pl.ds(r0 + j * GCH, GCH)])
        plsc.subcore_barrier()

        for g in range(3):
            wait_idx(g)
            pltpu.async_copy(y_hbm.at[sv.at[g]], rows.at[g], gsem)

        for g in range(16):
            step(g, g % NI, g % NR)

        def body(m, carry):
            g = m * NI
            for k in range(NI):
                step(g + k, k, k % NR)
            return carry

        lax.fori_loop(1, 7, body, 0)
        for g in range(112, NC_T):
            step(g, g % NI, g % NR)
        for g in range(121, 125):
            wait_rows(g % NR, ssem)

        plsc.subcore_barrier()
        NBK = RPT // GCH
        hbm_w = []
        for j in range(NBK):
            s = j % 2
            if j >= 2:
                hbm_w[j - 2].wait()
            pltpu.async_copy(acc.at[pl.ds(r0 + j * GCH, GCH)], rows.at[s], gsem).wait()
            hbm_w.append(pltpu.async_copy(
                rows.at[s],
                out_hbm.at[pl.ds(r0 + j * GCH, GCH), pl.ds(cid * D, D)], ssem))
        hbm_w[NBK - 2].wait()
        hbm_w[NBK - 1].wait()

    return agg_kernel(y, src, dst)


RBLK = 2048
EPAD = 327680
EBLK = EPAD // 5


def _tc0_body(ei_ref, x_ref, w_ref, src_ref, dst_ref, xw_ref):
    src_ref[...] = ei_ref[0, :]
    dst_ref[...] = ei_ref[1, :]
    xw_ref[...] = jnp.dot(x_ref[...], w_ref[...],
                          preferred_element_type=jnp.float32)


def _tc0(edge_index, x, W1):
    return pl.pallas_call(
        _tc0_body,
        grid=(5,),
        in_specs=[
            pl.BlockSpec((2, EBLK), lambda i: (0, i)),
            pl.BlockSpec((RBLK, 128), lambda i: (i, 0)),
            pl.BlockSpec((128, 64), lambda i: (0, 0)),
        ],
        out_specs=[
            pl.BlockSpec((EBLK,), lambda i: (i,)),
            pl.BlockSpec((EBLK,), lambda i: (i,)),
            pl.BlockSpec((RBLK, 64), lambda i: (i, 0)),
        ],
        out_shape=[
            jax.ShapeDtypeStruct((EPAD,), jnp.int32),
            jax.ShapeDtypeStruct((EPAD,), jnp.int32),
            jax.ShapeDtypeStruct((NN, 64), jnp.float32),
        ],
    )(edge_index, x, W1)


def _tc1_body(deg_ref, xw_ref, s_ref, y_ref):
    deg = deg_ref[0, :] + deg_ref[1, :] + 1.0
    s = lax.rsqrt(deg)[:, None]
    s_ref[...] = s
    y_ref[...] = xw_ref[...] * s


def _tc1(degp, xw):
    return pl.pallas_call(
        _tc1_body,
        grid=(5,),
        in_specs=[
            pl.BlockSpec((NCORE, RBLK), lambda i: (0, i)),
            pl.BlockSpec((RBLK, 64), lambda i: (i, 0)),
        ],
        out_specs=[
            pl.BlockSpec((RBLK, 1), lambda i: (i, 0)),
            pl.BlockSpec((RBLK, 64), lambda i: (i, 0)),
        ],
        out_shape=[
            jax.ShapeDtypeStruct((NN, 1), jnp.float32),
            jax.ShapeDtypeStruct((NN, 64), jnp.float32),
        ],
    )(degp, xw)


def _tc2_body(p_ref, y1_ref, s_ref, b1_ref, w2_ref, y2_ref):
    s = s_ref[...]
    agg = p_ref[:, :64] + p_ref[:, 64:128] + y1_ref[...]
    h1 = jnp.maximum(agg * s + b1_ref[...], 0.0)
    y2_ref[...] = jnp.dot(h1 * s, w2_ref[...],
                          preferred_element_type=jnp.float32)


def _tc2(p1, y1, s, b1, W2):
    return pl.pallas_call(
        _tc2_body,
        grid=(5,),
        in_specs=[
            pl.BlockSpec((RBLK, 128), lambda i: (i, 0)),
            pl.BlockSpec((RBLK, 64), lambda i: (i, 0)),
            pl.BlockSpec((RBLK, 1), lambda i: (i, 0)),
            pl.BlockSpec((64,), lambda i: (0,)),
            pl.BlockSpec((64, 32), lambda i: (0, 0)),
        ],
        out_specs=pl.BlockSpec((RBLK, 32), lambda i: (i, 0)),
        out_shape=jax.ShapeDtypeStruct((NN, 32), jnp.float32),
    )(p1, y1, s, b1, W2)


def _tc3_body(p_ref, y2_ref, s_ref, b2_ref, wc1_ref, bc1_ref,
              wc2_ref, bc2_ref, o_ref):
    s = s_ref[...]
    agg = p_ref[:, :32] + p_ref[:, 32:64] + y2_ref[...]
    h2 = jnp.maximum(agg * s + b2_ref[...], 0.0)
    h3 = jnp.maximum(
        jnp.dot(h2, wc1_ref[...], preferred_element_type=jnp.float32)
        + bc1_ref[...], 0.0)
    o_ref[...] = (jnp.dot(h3, wc2_ref[...], preferred_element_type=jnp.float32)
                  + bc2_ref[...])


def _tc3(p2, y2, s, b2, Wc1, bc1, Wc2, bc2):
    return pl.pallas_call(
        _tc3_body,
        grid=(5,),
        in_specs=[
            pl.BlockSpec((RBLK, 128), lambda i: (i, 0)),
            pl.BlockSpec((RBLK, 32), lambda i: (i, 0)),
            pl.BlockSpec((RBLK, 1), lambda i: (i, 0)),
            pl.BlockSpec((32,), lambda i: (0,)),
            pl.BlockSpec((32, 16), lambda i: (0, 0)),
            pl.BlockSpec((16,), lambda i: (0,)),
            pl.BlockSpec((16, 10), lambda i: (0, 0)),
            pl.BlockSpec((10,), lambda i: (0,)),
        ],
        out_specs=pl.BlockSpec((RBLK, 10), lambda i: (i, 0)),
        out_shape=jax.ShapeDtypeStruct((NN, 10), jnp.float32),
    )(p2, y2, s, b2, Wc1, bc1, Wc2, bc2)


def kernel(x, edge_index, batch, W1, b1, W2, b2, Wc1, bc1, Wc2, bc2):
    src, dst, xw = _tc0(edge_index, x, W1)
    degp = _sc_deg(dst)
    s, y1 = _tc1(degp, xw)
    p1 = _sc_agg(y1, src, dst, 64)
    y2 = _tc2(p1, y1, s, b1, W2)
    p2 = _sc_agg(y2, src, dst, 32)
    return _tc3(p2, y2, s, b2, Wc1, bc1, Wc2, bc2)


# speedup vs baseline: 53.9341x; 1.0003x over previous
import functools

import jax
import jax.numpy as jnp
from jax import lax
from jax.experimental import pallas as pl
from jax.experimental.pallas import tpu as pltpu
from jax.experimental.pallas import tpu_sc as plsc

NN = 10000
EE = 320000
NCORE = 2
NSUB = 16
NW = NCORE * NSUB
EPT = EE // NW
GCH = 80
NPAD = 10240
RPT = NPAD // NSUB
RPT_DEG = RPT
CH_D = 80


def _mesh():
    return plsc.VectorSubcoreMesh(
        core_axis_name="c", subcore_axis_name="s",
        num_cores=NCORE, num_subcores=NSUB)


def _zero_fill_1d(ref, nwords):
    z = jnp.zeros((16,), jnp.float32)

    def body(i, carry):
        ref[pl.ds(i * 16, 16)] = z
        return carry

    lax.fori_loop(0, nwords // 16, body, 0)


def _zero_fill_2d(ref, rows, cols):
    z = jnp.zeros((16,), jnp.float32)
    c16 = cols // 16

    def body(i, carry):
        r = i // c16
        c = i % c16
        ref[r, pl.ds(c * 16, 16)] = z
        return carry

    lax.fori_loop(0, rows * c16, body, 0)


def _sc_deg(dst):
    NID = 8
    NC_D = EPT // CH_D

    @functools.partial(
        pl.kernel,
        out_type=jax.ShapeDtypeStruct((NCORE, NPAD), jnp.float32),
        mesh=_mesh(),
        scratch_types=[
            pltpu.VMEM((NID, CH_D), jnp.int32),
            pltpu.VMEM((CH_D,), jnp.float32),
            pltpu.VMEM((RPT_DEG,), jnp.float32),
            pltpu.VMEM_SHARED((NPAD,), jnp.float32),
            pltpu.SemaphoreType.DMA,
            pltpu.SemaphoreType.DMA,
        ],
    )
    def deg_kernel(dst_hbm, out_hbm, dv, ones_v, zb, acc, isem, ssem):
        cid = lax.axis_index("c")
        sid = lax.axis_index("s")
        wid = sid * NCORE + cid
        base = wid * EPT
        one = jnp.ones((16,), jnp.float32)
        for k in range(CH_D // 16):
            ones_v[pl.ds(k * 16, 16)] = one

        def issue_idx(g, ig):
            off = pl.multiple_of(base + g * CH_D, 8)
            pltpu.async_copy(dst_hbm.at[pl.ds(off, CH_D)], dv.at[ig], isem)

        def wait_one(ig, sem):
            pltpu.make_async_copy(dst_hbm.at[pl.ds(0, CH_D)], dv.at[ig], sem).wait()

        def step(g, ig):
            wait_one(ig, isem)
            pltpu.async_copy(ones_v, acc.at[dv.at[ig]], ssem, add=True)
            if not isinstance(g, int) or g >= 4:
                wait_one((ig + 4) % NID, ssem)
            if not isinstance(g, int) or g + 4 <= NC_D - 1:
                issue_idx(g + 4, (ig + 4) % NID)

        for g in range(4):
            issue_idx(g, g)
        _zero_fill_1d(zb, RPT_DEG)
        pltpu.sync_copy(zb, acc.at[pl.ds(sid * RPT_DEG, RPT_DEG)])
        plsc.subcore_barrier()

        for g in range(8):
            step(g, g)

        def body(m, carry):
            for k in range(NID):
                step(m * NID + k, k)
            return carry

        lax.fori_loop(1, 15, body, 0)
        for g in range(120, 125):
            step(g, g % NID)
        for g in range(121, 125):
            wait_one(g % NID, ssem)

        plsc.subcore_barrier()
        pltpu.sync_copy(acc.at[pl.ds(sid * RPT_DEG, RPT_DEG)], zb)
        pltpu.sync_copy(zb, out_hbm.at[cid, pl.ds(sid * RPT_DEG, RPT_DEG)])

    return deg_kernel(dst)


def _sc_agg(y, src, dst, D):
    NC_T = EPT // GCH
    NI = 16
    NR = 8

    @functools.partial(
        pl.kernel,
        out_type=jax.ShapeDtypeStruct((NPAD, 128), jnp.float32),
        mesh=_mesh(),
        compiler_params=pltpu.CompilerParams(use_tc_tiling_on_sc=False),
        scratch_types=[
            pltpu.VMEM((NI, GCH), jnp.int32),
            pltpu.VMEM((NI, GCH), jnp.int32),
            pltpu.VMEM((NR, GCH, D), jnp.float32),
            pltpu.VMEM_SHARED((NPAD, D), jnp.float32),
            pltpu.SemaphoreType.DMA,
            pltpu.SemaphoreType.DMA,
            pltpu.SemaphoreType.DMA,
        ],
    )
    def agg_kernel(y_hbm, src_hbm, dst_hbm, out_hbm,
                   sv, dv, rows, acc, isem, gsem, ssem):
        cid = lax.axis_index("c")
        sid = lax.axis_index("s")
        wid = sid * NCORE + cid
        base = wid * EPT

        def issue_idx(g, ig):
            off = pl.multiple_of(base + g * GCH, 8)
            pltpu.async_copy(src_hbm.at[pl.ds(off, GCH)], sv.at[ig], isem)
            pltpu.async_copy(dst_hbm.at[pl.ds(off, GCH)], dv.at[ig], isem)

        def wait_idx(ig):
            pltpu.make_async_copy(src_hbm.at[pl.ds(0, GCH)], sv.at[ig], isem).wait()
            pltpu.make_async_copy(dst_hbm.at[pl.ds(0, GCH)], dv.at[ig], isem).wait()

        def wait_rows(rg, sem):
            pltpu.make_async_copy(y_hbm.at[pl.ds(0, GCH)], rows.at[rg], sem).wait()

        LAST = NC_T - 1

        def step(g, ig, rg):
            wait_rows(rg, gsem)
            pltpu.async_copy(rows.at[rg], acc.at[dv.at[ig]], ssem, add=True)
            if not isinstance(g, int) or g >= 4:
                wait_rows((rg + 4) % NR, ssem)
            if not isinstance(g, int) or g + 3 <= LAST:
                ig3 = (ig + 3) % NI
                wait_idx(ig3)
                pltpu.async_copy(y_hbm.at[sv.at[ig3]],
                                 rows.at[(rg + 3) % NR], gsem)
            if not isinstance(g, int) or g + 6 <= LAST:
                issue_idx(g + 6, (ig + 6) % NI)

        for g in range(6):
            issue_idx(g, g)
        _zero_fill_2d(rows.at[0], GCH, D)
        r0 = sid * RPT
        for j in range(RPT // GCH):
            pltpu.sync_copy(rows.at[0], acc.at[pl.ds(r0 + j * GCH, GCH)])
        plsc.subcore_barrier()

        for g in range(3):
            wait_idx(g)
            pltpu.async_copy(y_hbm.at[sv.at[g]], rows.at[g], gsem)

        for g in range(16):
            step(g, g % NI, g % NR)

        def body(m, carry):
            g = m * NI
            for k in range(NI):
                step(g + k, k, k % NR)
            return carry

        lax.fori_loop(1, 7, body, 0)
        for g in range(112, NC_T):
            step(g, g % NI, g % NR)
        for g in range(121, 125):
            wait_rows(g % NR, ssem)

        plsc.subcore_barrier()
        NBK = RPT // GCH
        hbm_w = []
        for j in range(NBK):
            s = j % 2
            if j >= 2:
                hbm_w[j - 2].wait()
            pltpu.async_copy(acc.at[pl.ds(r0 + j * GCH, GCH)], rows.at[s], gsem).wait()
            hbm_w.append(pltpu.async_copy(
                rows.at[s],
                out_hbm.at[pl.ds(r0 + j * GCH, GCH), pl.ds(cid * D, D)], ssem))
        hbm_w[NBK - 2].wait()
        hbm_w[NBK - 1].wait()

    return agg_kernel(y, src, dst)


RBLK = 2048
EPAD = 327680
EBLK = EPAD // 5


def _tc0_body(ei_ref, x_ref, w_ref, src_ref, dst_ref, xw_ref):
    src_ref[...] = ei_ref[0, :]
    dst_ref[...] = ei_ref[1, :]
    xw_ref[...] = jnp.dot(x_ref[...], w_ref[...],
                          preferred_element_type=jnp.float32)


def _tc0(edge_index, x, W1):
    return pl.pallas_call(
        _tc0_body,
        grid=(5,),
        in_specs=[
            pl.BlockSpec((2, EBLK), lambda i: (0, i)),
            pl.BlockSpec((RBLK, 128), lambda i: (i, 0)),
            pl.BlockSpec((128, 64), lambda i: (0, 0)),
        ],
        out_specs=[
            pl.BlockSpec((EBLK,), lambda i: (i,)),
            pl.BlockSpec((EBLK,), lambda i: (i,)),
            pl.BlockSpec((RBLK, 64), lambda i: (i, 0)),
        ],
        out_shape=[
            jax.ShapeDtypeStruct((EPAD,), jnp.int32),
            jax.ShapeDtypeStruct((EPAD,), jnp.int32),
            jax.ShapeDtypeStruct((NN, 64), jnp.float32),
        ],
    )(edge_index, x, W1)


def _tc1_body(deg_ref, xw_ref, s_ref, y_ref):
    deg = deg_ref[0, :] + deg_ref[1, :] + 1.0
    s = lax.rsqrt(deg)[:, None]
    s_ref[...] = s
    y_ref[...] = xw_ref[...] * s


def _tc1(degp, xw):
    return pl.pallas_call(
        _tc1_body,
        grid=(5,),
        in_specs=[
            pl.BlockSpec((NCORE, RBLK), lambda i: (0, i)),
            pl.BlockSpec((RBLK, 64), lambda i: (i, 0)),
        ],
        out_specs=[
            pl.BlockSpec((RBLK, 1), lambda i: (i, 0)),
            pl.BlockSpec((RBLK, 64), lambda i: (i, 0)),
        ],
        out_shape=[
            jax.ShapeDtypeStruct((NN, 1), jnp.float32),
            jax.ShapeDtypeStruct((NN, 64), jnp.float32),
        ],
    )(degp, xw)


def _tc2_body(p_ref, y1_ref, s_ref, b1_ref, w2_ref, y2_ref):
    s = s_ref[...]
    agg = p_ref[:, :64] + p_ref[:, 64:128] + y1_ref[...]
    h1 = jnp.maximum(agg * s + b1_ref[...], 0.0)
    y2_ref[...] = jnp.dot(h1 * s, w2_ref[...],
                          preferred_element_type=jnp.float32)


def _tc2(p1, y1, s, b1, W2):
    return pl.pallas_call(
        _tc2_body,
        grid=(5,),
        in_specs=[
            pl.BlockSpec((RBLK, 128), lambda i: (i, 0)),
            pl.BlockSpec((RBLK, 64), lambda i: (i, 0)),
            pl.BlockSpec((RBLK, 1), lambda i: (i, 0)),
            pl.BlockSpec((64,), lambda i: (0,)),
            pl.BlockSpec((64, 32), lambda i: (0, 0)),
        ],
        out_specs=pl.BlockSpec((RBLK, 32), lambda i: (i, 0)),
        out_shape=jax.ShapeDtypeStruct((NN, 32), jnp.float32),
    )(p1, y1, s, b1, W2)


def _tc3_body(p_ref, y2_ref, s_ref, b2_ref, wc1_ref, bc1_ref,
              wc2_ref, bc2_ref, o_ref):
    s = s_ref[...]
    agg = p_ref[:, :32] + p_ref[:, 32:64] + y2_ref[...]
    h2 = jnp.maximum(agg * s + b2_ref[...], 0.0)
    h3 = jnp.maximum(
        jnp.dot(h2, wc1_ref[...], preferred_element_type=jnp.float32)
        + bc1_ref[...], 0.0)
    o_ref[...] = (jnp.dot(h3, wc2_ref[...], preferred_element_type=jnp.float32)
                  + bc2_ref[...])


def _tc3(p2, y2, s, b2, Wc1, bc1, Wc2, bc2):
    return pl.pallas_call(
        _tc3_body,
        grid=(5,),
        in_specs=[
            pl.BlockSpec((RBLK, 128), lambda i: (i, 0)),
            pl.BlockSpec((RBLK, 32), lambda i: (i, 0)),
            pl.BlockSpec((RBLK, 1), lambda i: (i, 0)),
            pl.BlockSpec((32,), lambda i: (0,)),
            pl.BlockSpec((32, 16), lambda i: (0, 0)),
            pl.BlockSpec((16,), lambda i: (0,)),
            pl.BlockSpec((16, 10), lambda i: (0, 0)),
            pl.BlockSpec((10,), lambda i: (0,)),
        ],
        out_specs=pl.BlockSpec((RBLK, 10), lambda i: (i, 0)),
        out_shape=jax.ShapeDtypeStruct((NN, 10), jnp.float32),
    )(p2, y2, s, b2, Wc1, bc1, Wc2, bc2)


def kernel(x, edge_index, batch, W1, b1, W2, b2, Wc1, bc1, Wc2, bc2):
    src, dst, xw = _tc0(edge_index, x, W1)
    degp = _sc_deg(dst)
    s, y1 = _tc1(degp, xw)
    p1 = _sc_agg(y1, src, dst, 64)
    y2 = _tc2(p1, y1, s, b1, W2)
    p2 = _sc_agg(y2, src, dst, 32)
    return _tc3(p2, y2, s, b2, Wc1, bc1, Wc2, bc2)
